# baseline (speedup 1.0000x reference)
import functools

import jax
import jax.numpy as jnp
from jax import lax
from jax.experimental import pallas as pl
from jax.experimental.pallas import tpu as pltpu
from jax.experimental.pallas import tpu_sc as plsc

N_LOC = 1000000
N_USER = 100000
D_MODEL = 64
BATCH = 4096
HIST = 200

B_TOTAL = BATCH * HIST
NC = 2
NS = 16
NW = NC * NS
B_PER_W = B_TOTAL // NW
CHUNK = 512
NSUB = CHUNK // 128
SUP = 1024
NSUP = B_PER_W // SUP
NCHUNKS = B_PER_W // CHUNK


def _sc_gather_body(idx_hbm, table_hbm, out_hbm, idx_v, rows_v, sems):
    wid = lax.axis_index("s") * NC + lax.axis_index("c")
    gsem = (sems[0], sems[1])
    ssem = (sems[2], sems[3])
    iota16 = lax.iota(jnp.int32, 16)

    def idx_load(sup, slot):
        pltpu.sync_copy(idx_hbm.at[wid * NSUP + sup], idx_v.at[slot])

    def gather_start(slot, ss, half):
        for j in range(NSUB):
            pltpu.async_copy(
                table_hbm.at[idx_v.at[ss, half + j]],
                rows_v.at[slot, pl.ds(j * 128, 128)],
                gsem[slot],
            )

    def gather_wait(slot):
        pltpu.make_async_copy(
            table_hbm.at[pl.ds(0, CHUNK)], rows_v.at[slot], gsem[slot]
        ).wait()

    def scatter_start(k, slot):
        base = wid * B_PER_W + k * CHUNK
        pltpu.async_copy(rows_v.at[slot], out_hbm.at[pl.ds(base, CHUNK)], ssem[slot])

    def scatter_wait(slot):
        pltpu.make_async_copy(
            rows_v.at[slot], out_hbm.at[pl.ds(0, CHUNK)], ssem[slot]
        ).wait()

    def fixup(slot, ss, half):
        slot_vec = jnp.full((16,), slot, jnp.int32)
        zeros_f = jnp.zeros((16,), jnp.float32)
        for j in range(NSUB):
            def group(l, carry, j=j):
                iv = idx_v[ss, half + j, pl.ds(l * 16, 16)]

                nzero = plsc.all_reduce_population_count(iv == 0)

                @pl.when(nzero[0] > 0)
                def _():
                    pos = j * 128 + l * 16 + iota16
                    msk = iv == 0

                    def col_body(col, c2):
                        colv = jnp.full((16,), 0, jnp.int32) + col
                        plsc.store_scatter(
                            rows_v, (slot_vec, pos, colv), zeros_f, mask=msk
                        )
                        return c2

                    lax.fori_loop(0, D_MODEL, col_body, 0)

                return carry

            lax.fori_loop(0, 8, group, 0)

    def chunk_body(k, b, ss, next_ss, has_prev, has_next, load):
        gather_wait(b)
        if has_next:
            if has_prev:
                scatter_wait(1 - b)
            gather_start(1 - b, next_ss, (1 - b) * NSUB)
        fixup(b, ss, b * NSUB)
        if load is not None:
            sup, slot = load
            idx_load(sup, slot)
        scatter_start(k, b)

    idx_load(0, 0)
    gather_start(0, 0, 0)

    chunk_body(0, 0, 0, 0, False, True, (1, 1))
    chunk_body(1, 1, 0, 1, True, True, None)
    chunk_body(2, 0, 1, 1, True, True, (2, 0))
    chunk_body(3, 1, 1, 0, True, True, None)

    def quad(g, carry):
        k0 = 4 + 4 * g
        chunk_body(k0 + 0, 0, 0, 0, True, True, (3 + 2 * g, 1))
        chunk_body(k0 + 1, 1, 0, 1, True, True, None)
        chunk_body(k0 + 2, 0, 1, 1, True, True, (4 + 2 * g, 0))
        chunk_body(k0 + 3, 1, 1, 0, True, True, None)
        return carry

    lax.fori_loop(0, (NCHUNKS - 6) // 4, quad, 0)

    chunk_body(NCHUNKS - 2, 0, 0, 0, True, True, None)
    chunk_body(NCHUNKS - 1, 1, 0, 0, False, False, None)

    scatter_wait(0)
    scatter_wait(1)


@functools.cache
def _sc_gather():
    return pl.kernel(
        _sc_gather_body,
        out_type=jax.ShapeDtypeStruct((B_TOTAL, D_MODEL), jnp.float32),
        mesh=plsc.VectorSubcoreMesh(
            core_axis_name="c", subcore_axis_name="s", num_cores=NC, num_subcores=NS
        ),
        compiler_params=pltpu.CompilerParams(
            needs_layout_passes=False, use_tc_tiling_on_sc=False
        ),
        scratch_types=[
            pltpu.VMEM((2, 8, 128), jnp.int32),
            pltpu.VMEM((2, CHUNK, D_MODEL), jnp.float32),
            [pltpu.SemaphoreType.DMA] * 4,
        ],
    )


_U_BLK = 10000


def _tc_copy_body(u_ref, t_ref, uo_ref, to_ref):
    i = pl.program_id(0)
    gid = lax.broadcasted_iota(jnp.int32, (_U_BLK, 1), 0) + i * _U_BLK
    uo_ref[...] = jnp.where(gid == 0, 0.0, u_ref[...])

    @pl.when(i == 0)
    def _():
        to_ref[...] = t_ref[...]


def _tc_copy(user_table, time_table):
    return pl.pallas_call(
        _tc_copy_body,
        grid=(N_USER // _U_BLK,),
        in_specs=[
            pl.BlockSpec((_U_BLK, D_MODEL), lambda i: (i, 0)),
            pl.BlockSpec((24, D_MODEL), lambda i: (0, 0)),
        ],
        out_specs=[
            pl.BlockSpec((_U_BLK, D_MODEL), lambda i: (i, 0)),
            pl.BlockSpec((24, D_MODEL), lambda i: (0, 0)),
        ],
        out_shape=[
            jax.ShapeDtypeStruct((N_USER, D_MODEL), jnp.float32),
            jax.ShapeDtypeStruct((24, D_MODEL), jnp.float32),
        ],
    )(user_table, time_table)


def kernel(location_x, loc_table, user_table, time_table):
    idx3d = location_x.reshape(B_TOTAL // SUP, 8, 128)
    loc_flat = _sc_gather()(idx3d, loc_table)
    loc_embedded = loc_flat.reshape(BATCH, HIST, D_MODEL)
    user_embedded, timeslot_embedded = _tc_copy(user_table, time_table)
    return (loc_embedded, timeslot_embedded, user_embedded)
